# Initial kernel scaffold; baseline (speedup 1.0000x reference)
import jax
import jax.numpy as jnp
from jax.experimental import pallas as pl


def kernel(predict_0, predict_1, predict_2, targets):
    raise NotImplementedError("write your pallas kernel here")


# speedup vs baseline: 1.6821x; 1.6821x over previous
import functools

import jax
import jax.numpy as jnp
from jax import lax
from jax.experimental import pallas as pl
from jax.experimental.pallas import tpu as pltpu
from jax.experimental.pallas import tpu_sc as plsc

_B = 32
_N = 512
_M = 15 * _N
_NC, _NS = 2, 16
_NW = _NC * _NS
_CPT = _M // _NW
_VPT = _CPT // 16

_WS = (20, 40, 80)
_ANCHORS = (
    ((10.0, 13.0), (16.0, 30.0), (33.0, 23.0)),
    ((30.0, 61.0), (62.0, 45.0), (59.0, 119.0)),
    ((116.0, 90.0), (156.0, 198.0), (373.0, 326.0)),
)
_BAL = (4.0, 1.0, 0.4)


def _sigmoid(x):
    return 1.0 / (1.0 + jnp.exp(-x))


def _sc_body(p0_ref, p1_ref, p2_ref, tgt_ref,
             cls_out, valid_out, scal_out,
             tgt_v, idx_v, gbuf, vf_s, gtx_s, gty_s, w_s, h_s, aw_s, ah_s,
             cls_s, scal_s, sem):
    wid = lax.axis_index("s") * _NC + lax.axis_index("c")
    pltpu.sync_copy(tgt_ref, tgt_v)
    lanes = jnp.arange(16, dtype=jnp.int32)
    p_refs = (p0_ref, p1_ref, p2_ref)

    for l in range(3):
        w_l = _WS[l]
        wf = float(w_l)
        hw = w_l * w_l
        (a0w, a0h), (a1w, a1h), (a2w, a2h) = _ANCHORS[l]

        def cand_math(v, _):
            j = wid * _CPT + v * 16 + lanes
            t = lax.rem(j, _N)
            b = lax.div(j, _N)
            a = lax.rem(b, 3)
            g = lax.div(b, 3)
            t6 = t * 6
            img = plsc.load_gather(tgt_v, [t6]).astype(jnp.int32)
            cls = plsc.load_gather(tgt_v, [t6 + 1]).astype(jnp.int32)
            x = plsc.load_gather(tgt_v, [t6 + 2]) * wf
            y = plsc.load_gather(tgt_v, [t6 + 3]) * wf
            tw = plsc.load_gather(tgt_v, [t6 + 4]) * wf
            th = plsc.load_gather(tgt_v, [t6 + 5]) * wf
            aw = jnp.where(a == 0, a0w, jnp.where(a == 1, a1w, a2w))
            ah = jnp.where(a == 0, a0h, jnp.where(a == 1, a1h, a2h))
            rw = tw / aw
            rh = th / ah
            rmax = jnp.maximum(jnp.maximum(rw, 1.0 / rw),
                               jnp.maximum(rh, 1.0 / rh))
            vb = rmax < 4.0
            remx = lax.rem(x, 1.0)
            remy = lax.rem(y, 1.0)
            gm = ((g == 0)
                  | ((g == 1) & (remx < 0.5) & (x > 1.0))
                  | ((g == 2) & (remy < 0.5) & (y > 1.0))
                  | ((g == 3) & (remx > 0.5) & (x < wf - 1.0))
                  | ((g == 4) & (remy > 0.5) & (y < wf - 1.0)))
            valid = vb & gm
            offx = jnp.where(g == 1, 0.5, jnp.where(g == 3, -0.5, 0.0))
            offy = jnp.where(g == 2, 0.5, jnp.where(g == 4, -0.5, 0.0))
            gx = (x - offx).astype(jnp.int32)
            gy = (y - offy).astype(jnp.int32)
            base = (img * 255 + a * 85) * hw + gy * w_l + gx
            sl = pl.ds(v * 16, 16)
            idx_v[sl] = base
            vf_s[sl] = jnp.where(valid, 1.0, 0.0)
            gtx_s[sl] = x - gx.astype(jnp.float32)
            gty_s[sl] = y - gy.astype(jnp.float32)
            w_s[sl] = tw
            h_s[sl] = th
            aw_s[sl] = aw
            ah_s[sl] = ah
            cls_s[sl] = cls
            return 0

        lax.fori_loop(0, _VPT, cand_math, 0)

        def idx_row(c, _):
            for v in range(_VPT):
                prev = idx_v[pl.ds((c - 1) * _CPT + v * 16, 16)]
                idx_v[pl.ds(c * _CPT + v * 16, 16)] = prev + hw
            return 0

        lax.fori_loop(1, 85, idx_row, 0)
        pltpu.async_copy(p_refs[l].at[idx_v], gbuf, sem).wait()

        def giou_acc(v, carry):
            box_a, cnt_a, corr_a, pick_a = carry
            sl = pl.ds(v * 16, 16)
            op0 = gbuf[pl.ds(0 * _CPT + v * 16, 16)]
            op1 = gbuf[pl.ds(1 * _CPT + v * 16, 16)]
            op2 = gbuf[pl.ds(2 * _CPT + v * 16, 16)]
            op3 = gbuf[pl.ds(3 * _CPT + v * 16, 16)]
            op4 = gbuf[pl.ds(4 * _CPT + v * 16, 16)]
            vf = vf_s[sl]
            gtx = gtx_s[sl]
            gty = gty_s[sl]
            tw = w_s[sl]
            th = h_s[sl]
            aw = aw_s[sl]
            ah = ah_s[sl]
            sx = _sigmoid(op0) * 2.0 - 0.5
            sy = _sigmoid(op1) * 2.0 - 0.5
            swh = _sigmoid(op2) * 2.0
            sw = swh * swh * aw
            shh = _sigmoid(op3) * 2.0
            sh = shh * shh * ah
            a_xmin = sx - sw * 0.5
            a_xmax = sx + sw * 0.5
            a_ymin = sy - sh * 0.5
            a_ymax = sy + sh * 0.5
            b_xmin = gtx - tw * 0.5
            b_xmax = gtx + tw * 0.5
            b_ymin = gty - th * 0.5
            b_ymax = gty + th * 0.5
            iw = jnp.maximum(
                jnp.minimum(a_xmax, b_xmax) - jnp.maximum(a_xmin, b_xmin), 0.0)
            ih = jnp.maximum(
                jnp.minimum(a_ymax, b_ymax) - jnp.maximum(a_ymin, b_ymin), 0.0)
            inter = iw * ih
            union = ((a_xmax - a_xmin) * (a_ymax - a_ymin)
                     + (b_xmax - b_xmin) * (b_ymax - b_ymin) - inter)
            iou = inter / union
            cw = jnp.maximum(a_xmax, b_xmax) - jnp.minimum(a_xmin, b_xmin) + 1e-16
            ch = jnp.maximum(a_ymax, b_ymax) - jnp.minimum(a_ymin, b_ymin)
            carea = cw * ch + 1e-16
            giou = iou - (carea - union) / carea
            pick_idx = (cls_s[sl] + 5) * _CPT + v * 16 + lanes
            pick = plsc.load_gather(gbuf, [pick_idx])
            return (box_a + vf * (1.0 - giou),
                    cnt_a + vf,
                    corr_a + vf * op4 * jnp.maximum(giou, 0.0),
                    pick_a + vf * pick)

        zero = jnp.zeros((16,), jnp.float32)
        box_a, cnt_a, corr_a, pick_a = lax.fori_loop(
            0, _VPT, giou_acc, (zero, zero, zero, zero))
        scal_s[pl.ds(l * 64, 16)] = box_a
        scal_s[pl.ds(l * 64 + 16, 16)] = cnt_a
        scal_s[pl.ds(l * 64 + 32, 16)] = corr_a
        scal_s[pl.ds(l * 64 + 48, 16)] = pick_a
        pltpu.sync_copy(gbuf.at[pl.ds(5 * _CPT, 80 * _CPT)], cls_out.at[l, wid])
        pltpu.sync_copy(vf_s, valid_out.at[l, wid])

    pltpu.sync_copy(scal_s, scal_out.at[wid])


def _obj_body(p_ref, o_ref):
    a = pl.program_id(0)
    x = p_ref[...]
    s = jnp.sum(jnp.maximum(x, 0.0) + jnp.log1p(jnp.exp(-jnp.abs(x))))

    @pl.when(a == 0)
    def _():
        o_ref[...] = jnp.zeros((1, 1), jnp.float32)

    o_ref[...] = o_ref[...] + s


def _final_body(cls_ref, val_ref, scal_ref, o0_ref, o1_ref, o2_ref, out_ref):
    cls = cls_ref[...]
    vf = val_ref[...]
    sc = scal_ref[...]
    sp = jnp.maximum(cls, 0.0) + jnp.log1p(jnp.exp(-jnp.abs(cls)))
    objs = (o0_ref[0, 0], o1_ref[0, 0], o2_ref[0, 0])
    acc = 0.0
    for l in range(3):
        csum = jnp.sum(sp[l] * vf[l][:, None, :])
        box = jnp.sum(sc[:, l, 0, :])
        cnt = jnp.sum(sc[:, l, 1, :])
        corr = jnp.sum(sc[:, l, 2, :])
        pick = jnp.sum(sc[:, l, 3, :])
        denom = _B * 3.0 * float(_WS[l] * _WS[l])
        lobj = (objs[l] - corr) / denom * _BAL[l]
        lbox = jnp.where(cnt > 0, box / cnt, 0.0)
        lcls = jnp.where(cnt > 0, (csum - pick) / (cnt * 80.0), 0.0)
        acc = acc + 0.05 * lbox + lobj + 0.5 * lcls
    out_ref[...] = jnp.broadcast_to(acc * _B, (1, 1))


@jax.jit
def kernel(predict_0, predict_1, predict_2, targets):
    preds = (predict_0, predict_1, predict_2)
    flats = tuple(p.reshape(-1) for p in preds)

    sc_call = functools.partial(
        pl.kernel,
        out_type=[
            jax.ShapeDtypeStruct((3, _NW, 80 * _CPT), jnp.float32),
            jax.ShapeDtypeStruct((3, _NW, _CPT), jnp.float32),
            jax.ShapeDtypeStruct((_NW, 192), jnp.float32),
        ],
        mesh=plsc.VectorSubcoreMesh(core_axis_name="c", subcore_axis_name="s"),
        compiler_params=pltpu.CompilerParams(needs_layout_passes=False,
                                             use_tc_tiling_on_sc=False),
        scratch_types=[
            pltpu.VMEM((_N * 6,), jnp.float32),
            pltpu.VMEM((85 * _CPT,), jnp.int32),
            pltpu.VMEM((85 * _CPT,), jnp.float32),
            pltpu.VMEM((_CPT,), jnp.float32),
            pltpu.VMEM((_CPT,), jnp.float32),
            pltpu.VMEM((_CPT,), jnp.float32),
            pltpu.VMEM((_CPT,), jnp.float32),
            pltpu.VMEM((_CPT,), jnp.float32),
            pltpu.VMEM((_CPT,), jnp.float32),
            pltpu.VMEM((_CPT,), jnp.float32),
            pltpu.VMEM((_CPT,), jnp.int32),
            pltpu.VMEM((192,), jnp.float32),
            pltpu.SemaphoreType.DMA,
        ],
    )(_sc_body)
    cls_buf, valid_buf, scal_buf = sc_call(flats[0], flats[1], flats[2],
                                           targets.reshape(-1))

    obj_sums = []
    for l, p in enumerate(preds):
        h = w = _WS[l]
        obj_sums.append(pl.pallas_call(
            _obj_body,
            grid=(3,),
            in_specs=[pl.BlockSpec((_B, 1, h, w), lambda a: (0, 4 + 85 * a, 0, 0))],
            out_specs=pl.BlockSpec((1, 1), lambda a: (0, 0)),
            out_shape=jax.ShapeDtypeStruct((1, 1), jnp.float32),
        )(p))

    out = pl.pallas_call(
        _final_body,
        out_shape=jax.ShapeDtypeStruct((1, 1), jnp.float32),
    )(cls_buf.reshape(3, _NW, 80, _CPT), valid_buf,
      scal_buf.reshape(_NW, 3, 4, 16), *obj_sums)
    return out.reshape(1)
